# baseline (speedup 1.0000x reference)
import functools

import jax
import jax.numpy as jnp
from jax import lax
from jax.experimental import pallas as pl
from jax.experimental.pallas import tpu as pltpu
from jax.experimental.pallas import tpu_sc as plsc

N_TABLES = 26
NUM_EMB = 100000
EMB_DIM = 32

_NC, _NS = 2, 16
_NW = _NC * _NS
_LANE = 16

_CT = NUM_EMB // 128
_MAIN = _CT * 128
_MAINROWS = N_TABLES * _MAIN
_TAIL = NUM_EMB - _MAIN

_R = 64
_KK = _R * N_TABLES
_GROUPS = _KK // _LANE


def _linearize(wt, twflat):
    mesh = plsc.VectorSubcoreMesh(core_axis_name="c", subcore_axis_name="s")

    @functools.partial(
        pl.kernel,
        mesh=mesh,
        out_type=jax.ShapeDtypeStruct((N_TABLES * NUM_EMB * EMB_DIM,), jnp.float32),
        scratch_types=[
            pltpu.VMEM((2, 4, 8, 128), jnp.float32),
            pltpu.VMEM((4096,), jnp.float32),
            pltpu.VMEM((4096,), jnp.float32),
            pltpu.VMEM((1024,), jnp.float32),
            pltpu.SemaphoreType.DMA,
            pltpu.SemaphoreType.DMA,
            pltpu.SemaphoreType.DMA,
            pltpu.SemaphoreType.DMA,
        ],
        compiler_params=pltpu.CompilerParams(needs_layout_passes=False),
    )
    def ka(wt_hbm, tw_hbm, out_hbm, inb, outb0, outb1, tailv, gi0, gi1, wo0, wo1):
        outbs = (outb0, outb1)
        wid = lax.axis_index("s") * _NC + lax.axis_index("c")
        c_lo = (_CT * wid) // _NW
        c_hi = (_CT * (wid + 1)) // _NW
        nrounds = (c_hi - c_lo + 1) // 2

        gsems = (gi0, gi1)
        wsems = (wo0, wo1)
        iota32 = lax.iota(jnp.int32, _LANE) * EMB_DIM

        def block_in(t, c, j):
            for ti in range(4):
                pltpu.async_copy(
                    wt_hbm.at[t, pl.ds(ti * 8, 8), pl.ds(c * 128, 128)],
                    inb.at[j, ti],
                    gsems[j],
                )

        def block_in_wait(t, c, j):
            for ti in range(4):
                pltpu.make_async_copy(
                    wt_hbm.at[t, pl.ds(ti * 8, 8), pl.ds(c * 128, 128)],
                    inb.at[j, ti],
                    gsems[j],
                ).wait()

        def block_out(t, c, j):
            return pltpu.make_async_copy(
                outbs[j],
                out_hbm.at[pl.ds((t * _MAIN + c * 128) * EMB_DIM, 4096)],
                wsems[j],
            )

        def transpose(j):
            for ti in range(4):
                for dd in range(8):
                    d = ti * 8 + dd
                    for i in range(8):
                        v = inb[j, ti, dd, pl.ds(i * _LANE, _LANE)]
                        idxv = iota32 + (i * _LANE * EMB_DIM + d)
                        plsc.store_scatter(outbs[j], [idxv], v)

        @pl.loop(0, N_TABLES)
        def t_loop(t):
            @pl.loop(0, nrounds)
            def c_loop(r):
                for j in range(2):
                    c = c_lo + r * 2 + j

                    @pl.when(c < c_hi)
                    def _(t=t, c=c, j=j, r=r):
                        @pl.when((r > 0) | (t > 0))
                        def _():
                            block_out(t, c - 2, j).wait()

                        block_in(t, c, j)

                for j in range(2):
                    c = c_lo + r * 2 + j

                    @pl.when(c < c_hi)
                    def _(t=t, c=c, j=j):
                        block_in_wait(t, c, j)
                        transpose(j)
                        block_out(t, c, j).start()

        for j in range(2):
            @pl.when((c_lo + j) < c_hi)
            def _(j=j):
                block_out(0, c_lo, j).wait()

        @pl.when(wid < N_TABLES)
        def _():
            tb = _TAIL * EMB_DIM
            pltpu.sync_copy(tw_hbm.at[pl.ds(wid * tb, tb)], tailv)
            pltpu.sync_copy(
                tailv,
                out_hbm.at[pl.ds((_MAINROWS + wid * _TAIL) * EMB_DIM, tb)],
            )

    return ka(wt, twflat)


def _wide_embed(x_flat, w2, rep, offm, offt, *, total):
    bc = total // _NW
    rounds = bc // (2 * _R)

    mesh = plsc.VectorSubcoreMesh(core_axis_name="c", subcore_axis_name="s")

    @functools.partial(
        pl.kernel,
        mesh=mesh,
        out_type=jax.ShapeDtypeStruct((total * N_TABLES, EMB_DIM), jnp.float32),
        scratch_types=[
            pltpu.VMEM((bc,), jnp.int32),
            pltpu.VMEM((_KK,), jnp.int32),
            pltpu.VMEM((_KK,), jnp.int32),
            pltpu.VMEM((_KK,), jnp.int32),
            pltpu.VMEM((2, _KK), jnp.int32),
            pltpu.VMEM((2, _KK, EMB_DIM), jnp.float32),
            pltpu.SemaphoreType.DMA,
            pltpu.SemaphoreType.DMA,
            pltpu.SemaphoreType.DMA,
            pltpu.SemaphoreType.DMA,
        ],
        compiler_params=pltpu.CompilerParams(
            use_tc_tiling_on_sc=False, needs_layout_passes=False
        ),
    )
    def k(
        w_hbm, idx_hbm, rep_hbm, offm_hbm, offt_hbm, out_hbm,
        idx_v, rep_v, offm_v, offt_v, idx2_v, rows_v, g0, g1, w0, w1,
    ):
        wid = lax.axis_index("s") * _NC + lax.axis_index("c")
        base = wid * bc
        pltpu.sync_copy(idx_hbm.at[pl.ds(base, bc)], idx_v)
        pltpu.sync_copy(rep_hbm, rep_v)
        pltpu.sync_copy(offm_hbm, offm_v)
        pltpu.sync_copy(offt_hbm, offt_v)

        gsems = (g0, g1)
        wsems = (w0, w1)

        def build(j, c):
            rowbase = c * _R

            @pl.loop(0, _GROUPS)
            def _(gi):
                sl = pl.ds(gi * _LANE, _LANE)
                row = rowbase + rep_v[sl]
                xg = plsc.load_gather(idx_v, [row])
                idx2_v[j, sl] = jnp.where(
                    xg < _MAIN, xg + offm_v[sl], xg + offt_v[sl]
                )

        def gather(j):
            pltpu.async_copy(w_hbm.at[idx2_v.at[j]], rows_v.at[j], gsems[j])

        def gather_wait(j):
            pltpu.make_async_copy(
                w_hbm.at[idx2_v.at[j]], rows_v.at[j], gsems[j]
            ).wait()

        def wb(j, c):
            return pltpu.make_async_copy(
                rows_v.at[j],
                out_hbm.at[pl.ds((base + c * _R) * N_TABLES, _KK)],
                wsems[j],
            )

        @pl.loop(0, rounds)
        def round_loop(r):
            for j in range(2):
                @pl.when(r > 0)
                def _(j=j):
                    wb(j, (r - 1) * 2 + j).wait()

                build(j, r * 2 + j)
                gather(j)
            for j in range(2):
                gather_wait(j)
                wb(j, r * 2 + j).start()

        for j in range(2):
            wb(j, (rounds - 1) * 2 + j).wait()

    return k(w2, x_flat, rep, offm, offt)


def kernel(x, weight):
    B, T = x.shape
    total = B * T
    wt = jnp.transpose(weight, (0, 2, 1))
    twflat = weight[:, _MAIN:, :].reshape(N_TABLES * _TAIL * EMB_DIM)
    wflat = _linearize(wt, twflat)
    w2 = wflat.reshape(N_TABLES * NUM_EMB, EMB_DIM)

    karr = jnp.arange(_KK, dtype=jnp.int32)
    tk = karr % N_TABLES
    rep = karr // N_TABLES
    offm = tk * _MAIN
    offt = _MAINROWS + tk * _TAIL - _MAIN
    out = _wide_embed(x.reshape(total), w2, rep, offm, offt, total=total)
    return out.reshape(B, T, N_TABLES * EMB_DIM)


# speedup vs baseline: 1.0098x; 1.0098x over previous
import functools

import jax
import jax.numpy as jnp
from jax import lax
from jax.experimental import pallas as pl
from jax.experimental.pallas import tpu as pltpu
from jax.experimental.pallas import tpu_sc as plsc

N_TABLES = 26
NUM_EMB = 100000
EMB_DIM = 32

_NC, _NS = 2, 16
_NW = _NC * _NS
_LANE = 16

_CT = NUM_EMB // 128
_MAIN = _CT * 128
_MAINROWS = N_TABLES * _MAIN
_TAIL = NUM_EMB - _MAIN

_R = 64
_KK = _R * N_TABLES
_GROUPS = _KK // _LANE


def _linearize(wt, twflat):
    mesh = plsc.VectorSubcoreMesh(core_axis_name="c", subcore_axis_name="s")

    @functools.partial(
        pl.kernel,
        mesh=mesh,
        out_type=jax.ShapeDtypeStruct((N_TABLES * NUM_EMB * EMB_DIM,), jnp.float32),
        scratch_types=[
            pltpu.VMEM((2, 32, 2 * 128), jnp.float32),
            pltpu.VMEM((2 * 4096,), jnp.float32),
            pltpu.VMEM((2 * 4096,), jnp.float32),
            pltpu.VMEM((1024,), jnp.float32),
            pltpu.SemaphoreType.DMA,
            pltpu.SemaphoreType.DMA,
            pltpu.SemaphoreType.DMA,
            pltpu.SemaphoreType.DMA,
        ],
        compiler_params=pltpu.CompilerParams(needs_layout_passes=False),
    )
    def ka(wt_hbm, tw_hbm, out_hbm, inb, outb0, outb1, tailv, gi0, gi1, wo0, wo1):
        outbs = (outb0, outb1)
        wid = lax.axis_index("s") * _NC + lax.axis_index("c")
        nblk = _CT // 2
        b_lo = (nblk * wid) // _NW
        b_hi = (nblk * (wid + 1)) // _NW
        nrounds = (b_hi - b_lo + 1) // 2

        gsems = (gi0, gi1)
        wsems = (wo0, wo1)
        iota32 = lax.iota(jnp.int32, _LANE) * EMB_DIM

        def block_in(t, b, j):
            return pltpu.make_async_copy(
                wt_hbm.at[t, pl.ds(0, 32), pl.ds(b * 256, 256)],
                inb.at[j],
                gsems[j],
            )

        def block_out(t, b, j):
            return pltpu.make_async_copy(
                outbs[j],
                out_hbm.at[pl.ds((t * _MAIN + b * 256) * EMB_DIM, 8192)],
                wsems[j],
            )

        def transpose(j, groups):
            @pl.loop(0, 32)
            def _(d):
                @pl.loop(0, groups)
                def _(i):
                    v = inb[j, d, pl.ds(i * _LANE, _LANE)]
                    idxv = iota32 + (i * _LANE * EMB_DIM + d)
                    plsc.store_scatter(outbs[j], [idxv], v)

        @pl.loop(0, N_TABLES)
        def t_loop(t):
            @pl.loop(0, nrounds)
            def b_loop(r):
                for j in range(2):
                    b = b_lo + r * 2 + j

                    @pl.when(b < b_hi)
                    def _(t=t, b=b, j=j, r=r):
                        @pl.when((r > 0) | (t > 0))
                        def _():
                            block_out(t, b - 2, j).wait()

                        block_in(t, b, j).start()

                for j in range(2):
                    b = b_lo + r * 2 + j

                    @pl.when(b < b_hi)
                    def _(t=t, b=b, j=j):
                        block_in(t, b, j).wait()
                        transpose(j, 16)
                        block_out(t, b, j).start()

        for j in range(2):
            @pl.when((b_lo + j) < b_hi)
            def _(j=j):
                block_out(0, b_lo, j).wait()

        @pl.when(wid < N_TABLES)
        def _():
            pltpu.sync_copy(
                wt_hbm.at[wid, pl.ds(0, 32), pl.ds(_MAIN - 128, 128)],
                inb.at[0, pl.ds(0, 32), pl.ds(0, 128)],
            )
            transpose(0, 8)
            pltpu.sync_copy(
                outbs[0].at[pl.ds(0, 4096)],
                out_hbm.at[pl.ds((wid * _MAIN + _MAIN - 128) * EMB_DIM, 4096)],
            )

        @pl.when(wid < N_TABLES)
        def _():
            tb = _TAIL * EMB_DIM
            pltpu.sync_copy(tw_hbm.at[pl.ds(wid * tb, tb)], tailv)
            pltpu.sync_copy(
                tailv,
                out_hbm.at[pl.ds((_MAINROWS + wid * _TAIL) * EMB_DIM, tb)],
            )

    return ka(wt, twflat)


def _wide_embed(x_flat, w2, rep, offm, offt, *, total):
    bc = total // _NW
    rounds = bc // (2 * _R)

    mesh = plsc.VectorSubcoreMesh(core_axis_name="c", subcore_axis_name="s")

    @functools.partial(
        pl.kernel,
        mesh=mesh,
        out_type=jax.ShapeDtypeStruct((total * N_TABLES, EMB_DIM), jnp.float32),
        scratch_types=[
            pltpu.VMEM((bc,), jnp.int32),
            pltpu.VMEM((_KK,), jnp.int32),
            pltpu.VMEM((_KK,), jnp.int32),
            pltpu.VMEM((_KK,), jnp.int32),
            pltpu.VMEM((2, _KK), jnp.int32),
            pltpu.VMEM((2, _KK, EMB_DIM), jnp.float32),
            pltpu.SemaphoreType.DMA,
            pltpu.SemaphoreType.DMA,
            pltpu.SemaphoreType.DMA,
            pltpu.SemaphoreType.DMA,
        ],
        compiler_params=pltpu.CompilerParams(
            use_tc_tiling_on_sc=False, needs_layout_passes=False
        ),
    )
    def k(
        w_hbm, idx_hbm, rep_hbm, offm_hbm, offt_hbm, out_hbm,
        idx_v, rep_v, offm_v, offt_v, idx2_v, rows_v, g0, g1, w0, w1,
    ):
        wid = lax.axis_index("s") * _NC + lax.axis_index("c")
        base = wid * bc
        pltpu.sync_copy(idx_hbm.at[pl.ds(base, bc)], idx_v)
        pltpu.sync_copy(rep_hbm, rep_v)
        pltpu.sync_copy(offm_hbm, offm_v)
        pltpu.sync_copy(offt_hbm, offt_v)

        gsems = (g0, g1)
        wsems = (w0, w1)

        def build(j, c):
            rowbase = c * _R

            @pl.loop(0, _GROUPS)
            def _(gi):
                sl = pl.ds(gi * _LANE, _LANE)
                row = rowbase + rep_v[sl]
                xg = plsc.load_gather(idx_v, [row])
                idx2_v[j, sl] = jnp.where(
                    xg < _MAIN, xg + offm_v[sl], xg + offt_v[sl]
                )

        def gather(j):
            pltpu.async_copy(w_hbm.at[idx2_v.at[j]], rows_v.at[j], gsems[j])

        def gather_wait(j):
            pltpu.make_async_copy(
                w_hbm.at[idx2_v.at[j]], rows_v.at[j], gsems[j]
            ).wait()

        def wb(j, c):
            return pltpu.make_async_copy(
                rows_v.at[j],
                out_hbm.at[pl.ds((base + c * _R) * N_TABLES, _KK)],
                wsems[j],
            )

        @pl.loop(0, rounds)
        def round_loop(r):
            for j in range(2):
                @pl.when(r > 0)
                def _(j=j):
                    wb(j, (r - 1) * 2 + j).wait()

                build(j, r * 2 + j)
                gather(j)
            for j in range(2):
                gather_wait(j)
                wb(j, r * 2 + j).start()

        for j in range(2):
            wb(j, (rounds - 1) * 2 + j).wait()

    return k(w2, x_flat, rep, offm, offt)


def kernel(x, weight):
    B, T = x.shape
    total = B * T
    wt = jnp.transpose(weight, (0, 2, 1))
    twflat = weight[:, _MAIN:, :].reshape(N_TABLES * _TAIL * EMB_DIM)
    wflat = _linearize(wt, twflat)
    w2 = wflat.reshape(N_TABLES * NUM_EMB, EMB_DIM)

    karr = jnp.arange(_KK, dtype=jnp.int32)
    tk = karr % N_TABLES
    rep = karr // N_TABLES
    offm = tk * _MAIN
    offt = _MAINROWS + tk * _TAIL - _MAIN
    out = _wide_embed(x.reshape(total), w2, rep, offm, offt, total=total)
    return out.reshape(B, T, N_TABLES * EMB_DIM)


# speedup vs baseline: 1.0101x; 1.0002x over previous
import functools

import jax
import jax.numpy as jnp
from jax import lax
from jax.experimental import pallas as pl
from jax.experimental.pallas import tpu as pltpu
from jax.experimental.pallas import tpu_sc as plsc

N_TABLES = 26
NUM_EMB = 100000
EMB_DIM = 32

_NC, _NS = 2, 16
_NW = _NC * _NS
_LANE = 16

_CT = NUM_EMB // 128
_MAIN = _CT * 128
_MAINROWS = N_TABLES * _MAIN
_TAIL = NUM_EMB - _MAIN

_R = 64
_KK = _R * N_TABLES
_GROUPS = _KK // _LANE


def _linearize(wt, twflat):
    mesh = plsc.VectorSubcoreMesh(core_axis_name="c", subcore_axis_name="s")

    @functools.partial(
        pl.kernel,
        mesh=mesh,
        out_type=jax.ShapeDtypeStruct((N_TABLES * NUM_EMB * EMB_DIM,), jnp.float32),
        scratch_types=[
            pltpu.VMEM((2, 32, 2 * 128), jnp.float32),
            pltpu.VMEM((2 * 4096,), jnp.float32),
            pltpu.VMEM((2 * 4096,), jnp.float32),
            pltpu.VMEM((1024,), jnp.float32),
            pltpu.SemaphoreType.DMA,
            pltpu.SemaphoreType.DMA,
            pltpu.SemaphoreType.DMA,
            pltpu.SemaphoreType.DMA,
        ],
        compiler_params=pltpu.CompilerParams(needs_layout_passes=False),
    )
    def ka(wt_hbm, tw_hbm, out_hbm, inb, outb0, outb1, tailv, gi0, gi1, wo0, wo1):
        outbs = (outb0, outb1)
        wid = lax.axis_index("s") * _NC + lax.axis_index("c")
        nblk = _CT // 2
        b_lo = (nblk * wid) // _NW
        b_hi = (nblk * (wid + 1)) // _NW
        nrounds = (b_hi - b_lo + 1) // 2

        gsems = (gi0, gi1)
        wsems = (wo0, wo1)
        iota32 = lax.iota(jnp.int32, _LANE) * EMB_DIM

        def block_in(t, b, j):
            return pltpu.make_async_copy(
                wt_hbm.at[t, pl.ds(0, 32), pl.ds(b * 256, 256)],
                inb.at[j],
                gsems[j],
            )

        def block_out(t, b, j):
            return pltpu.make_async_copy(
                outbs[j],
                out_hbm.at[pl.ds((t * _MAIN + b * 256) * EMB_DIM, 8192)],
                wsems[j],
            )

        def transpose(j, groups):
            @pl.loop(0, 32)
            def _(d):
                base = iota32 + d
                for i in range(groups):
                    v = inb[j, d, pl.ds(i * _LANE, _LANE)]
                    plsc.store_scatter(outbs[j], [base + i * _LANE * EMB_DIM], v)

        @pl.loop(0, N_TABLES)
        def t_loop(t):
            @pl.loop(0, nrounds)
            def b_loop(r):
                for j in range(2):
                    b = b_lo + r * 2 + j

                    @pl.when(b < b_hi)
                    def _(t=t, b=b, j=j, r=r):
                        @pl.when((r > 0) | (t > 0))
                        def _():
                            block_out(t, b - 2, j).wait()

                        block_in(t, b, j).start()

                for j in range(2):
                    b = b_lo + r * 2 + j

                    @pl.when(b < b_hi)
                    def _(t=t, b=b, j=j):
                        block_in(t, b, j).wait()
                        transpose(j, 16)
                        block_out(t, b, j).start()

        for j in range(2):
            @pl.when((b_lo + j) < b_hi)
            def _(j=j):
                block_out(0, b_lo, j).wait()

        @pl.when(wid < N_TABLES)
        def _():
            pltpu.sync_copy(
                wt_hbm.at[wid, pl.ds(0, 32), pl.ds(_MAIN - 128, 128)],
                inb.at[0, pl.ds(0, 32), pl.ds(0, 128)],
            )
            transpose(0, 8)
            pltpu.sync_copy(
                outbs[0].at[pl.ds(0, 4096)],
                out_hbm.at[pl.ds((wid * _MAIN + _MAIN - 128) * EMB_DIM, 4096)],
            )

        @pl.when(wid < N_TABLES)
        def _():
            tb = _TAIL * EMB_DIM
            pltpu.sync_copy(tw_hbm.at[pl.ds(wid * tb, tb)], tailv)
            pltpu.sync_copy(
                tailv,
                out_hbm.at[pl.ds((_MAINROWS + wid * _TAIL) * EMB_DIM, tb)],
            )

    return ka(wt, twflat)


def _wide_embed(x_flat, w2, rep, offm, offt, *, total):
    bc = total // _NW
    rounds = bc // (2 * _R)

    mesh = plsc.VectorSubcoreMesh(core_axis_name="c", subcore_axis_name="s")

    @functools.partial(
        pl.kernel,
        mesh=mesh,
        out_type=jax.ShapeDtypeStruct((total * N_TABLES, EMB_DIM), jnp.float32),
        scratch_types=[
            pltpu.VMEM((bc,), jnp.int32),
            pltpu.VMEM((_KK,), jnp.int32),
            pltpu.VMEM((_KK,), jnp.int32),
            pltpu.VMEM((_KK,), jnp.int32),
            pltpu.VMEM((2, _KK), jnp.int32),
            pltpu.VMEM((2, _KK, EMB_DIM), jnp.float32),
            pltpu.SemaphoreType.DMA,
            pltpu.SemaphoreType.DMA,
            pltpu.SemaphoreType.DMA,
            pltpu.SemaphoreType.DMA,
        ],
        compiler_params=pltpu.CompilerParams(
            use_tc_tiling_on_sc=False, needs_layout_passes=False
        ),
    )
    def k(
        w_hbm, idx_hbm, rep_hbm, offm_hbm, offt_hbm, out_hbm,
        idx_v, rep_v, offm_v, offt_v, idx2_v, rows_v, g0, g1, w0, w1,
    ):
        wid = lax.axis_index("s") * _NC + lax.axis_index("c")
        base = wid * bc
        pltpu.sync_copy(idx_hbm.at[pl.ds(base, bc)], idx_v)
        pltpu.sync_copy(rep_hbm, rep_v)
        pltpu.sync_copy(offm_hbm, offm_v)
        pltpu.sync_copy(offt_hbm, offt_v)

        gsems = (g0, g1)
        wsems = (w0, w1)

        def build(j, c):
            rowbase = c * _R

            @pl.loop(0, _GROUPS)
            def _(gi):
                sl = pl.ds(gi * _LANE, _LANE)
                row = rowbase + rep_v[sl]
                xg = plsc.load_gather(idx_v, [row])
                idx2_v[j, sl] = jnp.where(
                    xg < _MAIN, xg + offm_v[sl], xg + offt_v[sl]
                )

        def gather(j):
            pltpu.async_copy(w_hbm.at[idx2_v.at[j]], rows_v.at[j], gsems[j])

        def gather_wait(j):
            pltpu.make_async_copy(
                w_hbm.at[idx2_v.at[j]], rows_v.at[j], gsems[j]
            ).wait()

        def wb(j, c):
            return pltpu.make_async_copy(
                rows_v.at[j],
                out_hbm.at[pl.ds((base + c * _R) * N_TABLES, _KK)],
                wsems[j],
            )

        @pl.loop(0, rounds)
        def round_loop(r):
            for j in range(2):
                @pl.when(r > 0)
                def _(j=j):
                    wb(j, (r - 1) * 2 + j).wait()

                build(j, r * 2 + j)
                gather(j)
            for j in range(2):
                gather_wait(j)
                wb(j, r * 2 + j).start()

        for j in range(2):
            wb(j, (rounds - 1) * 2 + j).wait()

    return k(w2, x_flat, rep, offm, offt)


def kernel(x, weight):
    B, T = x.shape
    total = B * T
    wt = jnp.transpose(weight, (0, 2, 1))
    twflat = weight[:, _MAIN:, :].reshape(N_TABLES * _TAIL * EMB_DIM)
    wflat = _linearize(wt, twflat)
    w2 = wflat.reshape(N_TABLES * NUM_EMB, EMB_DIM)

    karr = jnp.arange(_KK, dtype=jnp.int32)
    tk = karr % N_TABLES
    rep = karr // N_TABLES
    offm = tk * _MAIN
    offt = _MAINROWS + tk * _TAIL - _MAIN
    out = _wide_embed(x.reshape(total), w2, rep, offm, offt, total=total)
    return out.reshape(B, T, N_TABLES * EMB_DIM)


# speedup vs baseline: 2.4613x; 2.4367x over previous
import functools

import jax
import jax.numpy as jnp
from jax import lax
from jax.experimental import pallas as pl
from jax.experimental.pallas import tpu as pltpu
from jax.experimental.pallas import tpu_sc as plsc

N_TABLES = 26
NUM_EMB = 100000
EMB_DIM = 32

_NC, _NS = 2, 16
_NW = _NC * _NS
_LANE = 16

_CT = NUM_EMB // 128
_MAIN = _CT * 128
_MAINROWS = N_TABLES * _MAIN
_TAIL = NUM_EMB - _MAIN

_R = 64
_KK = _R * N_TABLES
_GROUPS = _KK // _LANE


def _linearize(wt, twflat):
    mesh = plsc.VectorSubcoreMesh(core_axis_name="c", subcore_axis_name="s")

    @functools.partial(
        pl.kernel,
        mesh=mesh,
        out_type=jax.ShapeDtypeStruct((N_TABLES * NUM_EMB * EMB_DIM,), jnp.float32),
        scratch_types=[
            pltpu.VMEM((2, 32, 2 * 128), jnp.float32),
            pltpu.VMEM((2 * 4096,), jnp.float32),
            pltpu.VMEM((2 * 4096,), jnp.float32),
            pltpu.VMEM((1024,), jnp.float32),
            pltpu.SemaphoreType.DMA,
            pltpu.SemaphoreType.DMA,
            pltpu.SemaphoreType.DMA,
            pltpu.SemaphoreType.DMA,
        ],
        compiler_params=pltpu.CompilerParams(needs_layout_passes=False),
    )
    def ka(wt_hbm, tw_hbm, out_hbm, inb, outb0, outb1, tailv, gi0, gi1, wo0, wo1):
        outbs = (outb0, outb1)
        wid = lax.axis_index("s") * _NC + lax.axis_index("c")
        nblk = _CT // 2
        b_lo = (nblk * wid) // _NW
        b_hi = (nblk * (wid + 1)) // _NW
        nrounds = (b_hi - b_lo + 1) // 2

        gsems = (gi0, gi1)
        wsems = (wo0, wo1)
        iota32 = lax.iota(jnp.int32, _LANE) * EMB_DIM

        def block_in(t, b, j):
            return pltpu.make_async_copy(
                wt_hbm.at[t, pl.ds(0, 32), pl.ds(b * 256, 256)],
                inb.at[j],
                gsems[j],
            )

        def block_out(t, b, j):
            return pltpu.make_async_copy(
                outbs[j],
                out_hbm.at[pl.ds((t * _MAIN + b * 256) * EMB_DIM, 8192)],
                wsems[j],
            )

        def transpose(j, groups):
            @functools.partial(plsc.parallel_loop, 0, 32, unroll=4)
            def _(d):
                base = iota32 + d
                for i in range(groups):
                    v = inb[j, d, pl.ds(i * _LANE, _LANE)]
                    plsc.store_scatter(outbs[j], [base + i * _LANE * EMB_DIM], v)

        @pl.loop(0, N_TABLES)
        def t_loop(t):
            @pl.loop(0, nrounds)
            def b_loop(r):
                for j in range(2):
                    b = b_lo + r * 2 + j

                    @pl.when(b < b_hi)
                    def _(t=t, b=b, j=j, r=r):
                        @pl.when((r > 0) | (t > 0))
                        def _():
                            block_out(t, b - 2, j).wait()

                        block_in(t, b, j).start()

                for j in range(2):
                    b = b_lo + r * 2 + j

                    @pl.when(b < b_hi)
                    def _(t=t, b=b, j=j):
                        block_in(t, b, j).wait()
                        transpose(j, 16)
                        block_out(t, b, j).start()

        for j in range(2):
            @pl.when((b_lo + j) < b_hi)
            def _(j=j):
                block_out(0, b_lo, j).wait()

        @pl.when(wid < N_TABLES)
        def _():
            pltpu.sync_copy(
                wt_hbm.at[wid, pl.ds(0, 32), pl.ds(_MAIN - 128, 128)],
                inb.at[0, pl.ds(0, 32), pl.ds(0, 128)],
            )
            transpose(0, 8)
            pltpu.sync_copy(
                outbs[0].at[pl.ds(0, 4096)],
                out_hbm.at[pl.ds((wid * _MAIN + _MAIN - 128) * EMB_DIM, 4096)],
            )

        @pl.when(wid < N_TABLES)
        def _():
            tb = _TAIL * EMB_DIM
            pltpu.sync_copy(tw_hbm.at[pl.ds(wid * tb, tb)], tailv)
            pltpu.sync_copy(
                tailv,
                out_hbm.at[pl.ds((_MAINROWS + wid * _TAIL) * EMB_DIM, tb)],
            )

    return ka(wt, twflat)


def _wide_embed(x_flat, w2, rep, offm, offt, *, total):
    bc = total // _NW
    rounds = bc // (2 * _R)

    mesh = plsc.VectorSubcoreMesh(core_axis_name="c", subcore_axis_name="s")

    @functools.partial(
        pl.kernel,
        mesh=mesh,
        out_type=jax.ShapeDtypeStruct((total * N_TABLES, EMB_DIM), jnp.float32),
        scratch_types=[
            pltpu.VMEM((bc,), jnp.int32),
            pltpu.VMEM((_KK,), jnp.int32),
            pltpu.VMEM((_KK,), jnp.int32),
            pltpu.VMEM((_KK,), jnp.int32),
            pltpu.VMEM((2, _KK), jnp.int32),
            pltpu.VMEM((2, _KK, EMB_DIM), jnp.float32),
            pltpu.SemaphoreType.DMA,
            pltpu.SemaphoreType.DMA,
            pltpu.SemaphoreType.DMA,
            pltpu.SemaphoreType.DMA,
        ],
        compiler_params=pltpu.CompilerParams(
            use_tc_tiling_on_sc=False, needs_layout_passes=False
        ),
    )
    def k(
        w_hbm, idx_hbm, rep_hbm, offm_hbm, offt_hbm, out_hbm,
        idx_v, rep_v, offm_v, offt_v, idx2_v, rows_v, g0, g1, w0, w1,
    ):
        wid = lax.axis_index("s") * _NC + lax.axis_index("c")
        base = wid * bc
        pltpu.sync_copy(idx_hbm.at[pl.ds(base, bc)], idx_v)
        pltpu.sync_copy(rep_hbm, rep_v)
        pltpu.sync_copy(offm_hbm, offm_v)
        pltpu.sync_copy(offt_hbm, offt_v)

        gsems = (g0, g1)
        wsems = (w0, w1)

        def build(j, c):
            rowbase = c * _R

            @pl.loop(0, _GROUPS)
            def _(gi):
                sl = pl.ds(gi * _LANE, _LANE)
                row = rowbase + rep_v[sl]
                xg = plsc.load_gather(idx_v, [row])
                idx2_v[j, sl] = jnp.where(
                    xg < _MAIN, xg + offm_v[sl], xg + offt_v[sl]
                )

        def gather(j):
            pltpu.async_copy(w_hbm.at[idx2_v.at[j]], rows_v.at[j], gsems[j])

        def gather_wait(j):
            pltpu.make_async_copy(
                w_hbm.at[idx2_v.at[j]], rows_v.at[j], gsems[j]
            ).wait()

        def wb(j, c):
            return pltpu.make_async_copy(
                rows_v.at[j],
                out_hbm.at[pl.ds((base + c * _R) * N_TABLES, _KK)],
                wsems[j],
            )

        @pl.loop(0, rounds)
        def round_loop(r):
            for j in range(2):
                @pl.when(r > 0)
                def _(j=j):
                    wb(j, (r - 1) * 2 + j).wait()

                build(j, r * 2 + j)
                gather(j)
            for j in range(2):
                gather_wait(j)
                wb(j, r * 2 + j).start()

        for j in range(2):
            wb(j, (rounds - 1) * 2 + j).wait()

    return k(w2, x_flat, rep, offm, offt)


def kernel(x, weight):
    B, T = x.shape
    total = B * T
    wt = jnp.transpose(weight, (0, 2, 1))
    twflat = weight[:, _MAIN:, :].reshape(N_TABLES * _TAIL * EMB_DIM)
    wflat = _linearize(wt, twflat)
    w2 = wflat.reshape(N_TABLES * NUM_EMB, EMB_DIM)

    karr = jnp.arange(_KK, dtype=jnp.int32)
    tk = karr % N_TABLES
    rep = karr // N_TABLES
    offm = tk * _MAIN
    offt = _MAINROWS + tk * _TAIL - _MAIN
    out = _wide_embed(x.reshape(total), w2, rep, offm, offt, total=total)
    return out.reshape(B, T, N_TABLES * EMB_DIM)
